# baseline (speedup 1.0000x reference)
import jax
import jax.numpy as jnp
from jax import lax
from jax.experimental import pallas as pl
from jax.experimental.pallas import tpu as pltpu
from jax.experimental.pallas import tpu_sc as plsc

N_NODES = 10000
N_PAD = 10240
DIM = 128
BLK = 512

NC = 2
NS = 16
NW = NC * NS
E_PAD = 327680
EW = E_PAD // NW
EW0 = 15360
EW1 = 5120
EWM = max(EW0, EW1)
E_ALLOC = NS * (EW0 + EW1) + max(0, EW0 - EW1)
C2 = 128
EWB = 1024
CPP = EWB // C2
ROWS_T = N_PAD // NS
C = 128
NEG = -1e30


def _leaky(x):
    return jnp.where(x >= 0, x, 0.01 * x)


def _tc_node_body(h_ref, wfc_ref, wcat_ref, z_ref, s_ref):
    z = _leaky(jnp.dot(h_ref[...], wfc_ref[...].T,
                       preferred_element_type=jnp.float32))
    z_ref[...] = z
    s_ref[...] = jnp.dot(z, wcat_ref[...].T,
                         preferred_element_type=jnp.float32)


def _tc_node_transform(h_pad, W_fc, Wcat_scaled):
    return pl.pallas_call(
        _tc_node_body,
        grid=(N_PAD // BLK,),
        in_specs=[
            pl.BlockSpec((BLK, DIM), lambda i: (i, 0)),
            pl.BlockSpec((DIM, DIM), lambda i: (0, 0)),
            pl.BlockSpec((2, DIM), lambda i: (0, 0)),
        ],
        out_specs=[
            pl.BlockSpec((BLK, DIM), lambda i: (i, 0)),
            pl.BlockSpec((BLK, 2), lambda i: (i, 0)),
        ],
        out_shape=[
            jax.ShapeDtypeStruct((N_PAD, DIM), jnp.float32),
            jax.ShapeDtypeStruct((N_PAD, 2), jnp.float32),
        ],
    )(h_pad, W_fc, Wcat_scaled)


def _sc_logits_body(src_h, dst_h, u_h, s1_h, s2_h, wm_h,
                    ne_h, mpart_h,
                    s1_v, s2_v, mmax_v, src_v, dst_v, u_v, ne_v, wm_v,
                    macc_v, mtmp_v, shared_max):
    cid = lax.axis_index("c")
    sid = lax.axis_index("s")
    wid = cid * NS + sid
    base = wid * EW
    pltpu.sync_copy(s1_h, s1_v)
    pltpu.sync_copy(s2_h, s2_v)
    pltpu.sync_copy(wm_h, wm_v)
    pltpu.sync_copy(src_h.at[pl.ds(base, EW)], src_v)
    pltpu.sync_copy(dst_h.at[pl.ds(base, EW)], dst_v)
    pltpu.sync_copy(u_h.at[pl.ds(base, EW)], u_v)
    wm = wm_v[...]

    def init(i, carry):
        mmax_v[pl.ds(i * 16, 16)] = jnp.full((16,), NEG, jnp.float32)
        return carry
    lax.fori_loop(0, N_PAD // 16, init, 0)

    def grp(g, carry):
        sl = pl.ds(g * 16, 16)
        src16 = src_v[sl]
        dst16 = dst_v[sl]
        u16 = u_v[sl]
        a = plsc.load_gather(s1_v, [src16]) + plsc.load_gather(s2_v, [dst16])
        t = a * u16
        ne = wm * jnp.where(t >= 0, t, 0.01 * t)
        ne_v[sl] = ne

        def rtry(r, c2):
            cur = plsc.load_gather(mmax_v, [dst16])
            plsc.store_scatter(mmax_v, [dst16], ne, mask=ne > cur)
            return c2
        lax.fori_loop(0, 6, rtry, 0)
        return carry
    lax.fori_loop(0, EW // 16, grp, 0)
    pltpu.sync_copy(ne_v, ne_h.at[pl.ds(base, EW)])

    pltpu.sync_copy(mmax_v, shared_max.at[sid])
    plsc.subcore_barrier()
    rsl = pl.ds(sid * ROWS_T, ROWS_T)
    pltpu.sync_copy(shared_max.at[0, rsl], macc_v)

    def mrg(t, carry):
        pltpu.sync_copy(shared_max.at[t, rsl], mtmp_v)

        def mx(i, c2):
            isl = pl.ds(i * 16, 16)
            macc_v[isl] = jnp.maximum(macc_v[isl], mtmp_v[isl])
            return c2
        lax.fori_loop(0, ROWS_T // 16, mx, 0)
        return carry
    lax.fori_loop(1, NS, mrg, 0)
    pltpu.sync_copy(macc_v, mpart_h.at[cid, rsl])


def _sc_agg_body(src_h, dst2_h, ne_h, mpart_h, z_h,
                 acc_h, den_h,
                 m_v, src_v, ne_v, dst2_v, u_c0, u_c1, idx0, idx1,
                 rows0, rows1, zden_v, acc_sp, den_sp, sem0, sem1):
    cid = lax.axis_index("c")
    sid = lax.axis_index("s")
    base = jnp.where(cid == 0, sid * EW0, NS * EW0 + sid * EW1)
    npass = jnp.where(cid == 0, EW0 // EWB, EW1 // EWB)

    pltpu.sync_copy(mpart_h.at[0], m_v)

    def mpass(q, carry):
        pltpu.sync_copy(mpart_h.at[1, pl.ds(q * EWB, EWB)], ne_v)

        def mx(i, c2):
            m_v[pl.ds(q * EWB + i * 16, 16)] = jnp.maximum(
                m_v[pl.ds(q * EWB + i * 16, 16)], ne_v[pl.ds(i * 16, 16)])
            return c2
        lax.fori_loop(0, EWB // 16, mx, 0)
        return carry
    lax.fori_loop(0, N_PAD // EWB, mpass, 0)

    def z16(i, carry):
        for j in range(DIM // 16):
            rows0[i, pl.ds(j * 16, 16)] = jnp.zeros((16,), jnp.float32)
        return carry
    lax.fori_loop(0, C2, z16, 0)

    def zden(i, carry):
        zden_v[pl.ds(i * 16, 16)] = jnp.zeros((16,), jnp.float32)
        return carry
    lax.fori_loop(0, ROWS_T // 16, zden, 0)

    def zacc(i, carry):
        pltpu.sync_copy(rows0, acc_sp.at[pl.ds(sid * ROWS_T + i * C2, C2)])
        return carry
    lax.fori_loop(0, ROWS_T // C2, zacc, 0)
    pltpu.sync_copy(zden_v, den_sp.at[pl.ds(sid * ROWS_T, ROWS_T)])
    plsc.subcore_barrier()

    bufs = ((u_c0, rows0, sem0, idx0), (u_c1, rows1, sem1, idx1))

    def issue_gather(c, b):
        _, rows_b, sem_b, _ = bufs[b]
        pltpu.async_copy(z_h.at[src_v.at[pl.ds(c * C2, C2)]], rows_b, sem_b)

    def pas(p, carry):
        pbase = pl.multiple_of(base + p * EWB, EWB)
        prow = pl.multiple_of(pbase // C2, CPP)
        pltpu.sync_copy(src_h.at[pl.ds(pbase, EWB)], src_v)
        pltpu.sync_copy(ne_h.at[pl.ds(pbase, EWB)], ne_v)
        pltpu.sync_copy(dst2_h.at[pl.ds(prow, CPP)], dst2_v)
        for b in (0, 1):
            issue_gather(b, b)

        def outer(c2i, carry2):
            for b in (0, 1):
                u_b, rows_b, sem_b, idx_b = bufs[b]
                c = 2 * c2i + b

                def grp(g, c3):
                    sl = pl.ds(g * 16, 16)
                    dst16 = dst2_v[c, sl]
                    idx_b[sl] = dst16
                    m16 = plsc.load_gather(m_v, [dst16])
                    u_b[sl] = jnp.exp(ne_v[pl.ds(c * C2 + g * 16, 16)] - m16)
                    return c3
                lax.fori_loop(0, C2 // 16, grp, 0)

                pltpu.make_async_copy(z_h.at[pl.ds(0, C2)], rows_b,
                                      sem_b).wait()

                def scale(e, c3):
                    ue = plsc.load_gather(u_b,
                                          [jnp.zeros((16,), jnp.int32) + e])
                    for j in range(DIM // 16):
                        jsl = pl.ds(j * 16, 16)
                        rows_b[e, jsl] = rows_b[e, jsl] * ue
                    return c3
                lax.fori_loop(0, C2, scale, 0)

                pltpu.sync_copy(u_b, den_sp.at[idx_b], add=True)
                pltpu.sync_copy(rows_b, acc_sp.at[idx_b], add=True)

                @pl.when(c + 2 < CPP)
                def _():
                    issue_gather(c + 2, b)
            return carry2
        lax.fori_loop(0, CPP // 2, outer, 0)
        return carry
    lax.fori_loop(0, npass, pas, 0)
    plsc.subcore_barrier()

    rsl = pl.ds(sid * ROWS_T, ROWS_T)
    pltpu.sync_copy(acc_sp.at[rsl], acc_h.at[cid, rsl])
    pltpu.sync_copy(den_sp.at[rsl], den_h.at[cid, rsl])


def _tc_merge_body(acc_ref, den_ref, o_ref):
    num = acc_ref[0] + acc_ref[1]
    den = den_ref[0] + den_ref[1]
    d = jnp.where(den > 0, den, 1.0)
    o_ref[...] = _leaky(num / d)


def _tc_merge(acc, den3):
    return pl.pallas_call(
        _tc_merge_body,
        grid=(N_PAD // BLK,),
        in_specs=[
            pl.BlockSpec((NC, BLK, DIM), lambda i: (0, i, 0)),
            pl.BlockSpec((NC, BLK, 1), lambda i: (0, i, 0)),
        ],
        out_specs=pl.BlockSpec((BLK, DIM), lambda i: (i, 0)),
        out_shape=jax.ShapeDtypeStruct((N_PAD, DIM), jnp.float32),
    )(acc, den3)


def kernel(h, edge_index, edge_attr, W_fc, W_attn, W_edge, W_m):
    E = edge_index.shape[1]
    src = edge_index[0].astype(jnp.int32)
    dst = edge_index[1].astype(jnp.int32)
    u = edge_attr[:, 0]
    pad = E_ALLOC - E
    src = jnp.concatenate([src, jnp.zeros((pad,), jnp.int32)])
    pad_dst = N_NODES + (jnp.arange(pad, dtype=jnp.int32) % (N_PAD - N_NODES))
    dst = jnp.concatenate([dst, pad_dst])
    u = jnp.concatenate([u, jnp.zeros((pad,), jnp.float32)])

    h_pad = jnp.pad(h, ((0, N_PAD - N_NODES), (0, 0)))
    Wcat_scaled = W_attn.reshape(2, DIM) * W_edge[0, 0]
    z, s = _tc_node_transform(h_pad, W_fc, Wcat_scaled)
    s1 = s[:, 0] + 0.0
    s2 = s[:, 1] + 0.0
    wm16 = jnp.full((16,), W_m[0, 0], jnp.float32)

    mesh = plsc.VectorSubcoreMesh(core_axis_name="c", subcore_axis_name="s")

    sc_params = pltpu.CompilerParams(needs_layout_passes=False)
    sc1 = pl.kernel(
        _sc_logits_body,
        mesh=mesh,
        compiler_params=sc_params,
        out_type=[
            jax.ShapeDtypeStruct((E_PAD,), jnp.float32),
            jax.ShapeDtypeStruct((NC, N_PAD), jnp.float32),
        ],
        scratch_types=[
            pltpu.VMEM((N_PAD,), jnp.float32),
            pltpu.VMEM((N_PAD,), jnp.float32),
            pltpu.VMEM((N_PAD,), jnp.float32),
            pltpu.VMEM((EW,), jnp.int32),
            pltpu.VMEM((EW,), jnp.int32),
            pltpu.VMEM((EW,), jnp.float32),
            pltpu.VMEM((EW,), jnp.float32),
            pltpu.VMEM((16,), jnp.float32),
            pltpu.VMEM((ROWS_T,), jnp.float32),
            pltpu.VMEM((ROWS_T,), jnp.float32),
            pltpu.VMEM_SHARED((NS, N_PAD), jnp.float32),
        ],
    )
    ne, mpart = sc1(src, dst, u, s1, s2, wm16)
    ne_pad = jnp.pad(ne, (0, E_ALLOC - E_PAD)) if E_ALLOC > E_PAD else ne

    sc2 = pl.kernel(
        _sc_agg_body,
        mesh=mesh,
        compiler_params=sc_params,
        out_type=[
            jax.ShapeDtypeStruct((NC, N_PAD, DIM), jnp.float32),
            jax.ShapeDtypeStruct((NC, N_PAD), jnp.float32),
        ],
        scratch_types=[
            pltpu.VMEM((N_PAD,), jnp.float32),
            pltpu.VMEM((EWB,), jnp.int32),
            pltpu.VMEM((EWB,), jnp.float32),
            pltpu.VMEM((CPP, C2), jnp.int32),
            pltpu.VMEM((C2,), jnp.float32),
            pltpu.VMEM((C2,), jnp.float32),
            pltpu.VMEM((C2,), jnp.int32),
            pltpu.VMEM((C2,), jnp.int32),
            pltpu.VMEM((C2, DIM), jnp.float32),
            pltpu.VMEM((C2, DIM), jnp.float32),
            pltpu.VMEM((ROWS_T,), jnp.float32),
            pltpu.VMEM_SHARED((N_PAD, DIM), jnp.float32),
            pltpu.VMEM_SHARED((N_PAD,), jnp.float32),
            pltpu.SemaphoreType.DMA,
            pltpu.SemaphoreType.DMA,
        ],
    )
    dst2 = dst.reshape(E_ALLOC // C2, C2)
    acc, den = sc2(src, dst2, ne_pad, mpart, z)

    out = _tc_merge(acc, den.reshape(NC, N_PAD, 1))
    return out[:N_NODES]


# speedup vs baseline: 1.0302x; 1.0302x over previous
import jax
import jax.numpy as jnp
from jax import lax
from jax.experimental import pallas as pl
from jax.experimental.pallas import tpu as pltpu
from jax.experimental.pallas import tpu_sc as plsc

N_NODES = 10000
N_PAD = 10240
DIM = 128
BLK = 512

NC = 2
NS = 16
NW = NC * NS
E_PAD = 327680
EW = E_PAD // NW
EW0 = 15360
EW1 = 5120
EWM = max(EW0, EW1)
E_ALLOC = NS * (EW0 + EW1) + max(0, EW0 - EW1)
C2 = 64
EWB = 2560
CPP = EWB // C2
ROWS_T = N_PAD // NS
C = 128
NEG = -1e30


def _leaky(x):
    return jnp.where(x >= 0, x, 0.01 * x)


def _tc_node_body(h_ref, wfc_ref, wcat_ref, z_ref, s_ref):
    z = _leaky(jnp.dot(h_ref[...], wfc_ref[...].T,
                       preferred_element_type=jnp.float32))
    z_ref[...] = z
    s_ref[...] = jnp.dot(z, wcat_ref[...].T,
                         preferred_element_type=jnp.float32)


def _tc_node_transform(h_pad, W_fc, Wcat_scaled):
    return pl.pallas_call(
        _tc_node_body,
        grid=(N_PAD // BLK,),
        in_specs=[
            pl.BlockSpec((BLK, DIM), lambda i: (i, 0)),
            pl.BlockSpec((DIM, DIM), lambda i: (0, 0)),
            pl.BlockSpec((2, DIM), lambda i: (0, 0)),
        ],
        out_specs=[
            pl.BlockSpec((BLK, DIM), lambda i: (i, 0)),
            pl.BlockSpec((BLK, 2), lambda i: (i, 0)),
        ],
        out_shape=[
            jax.ShapeDtypeStruct((N_PAD, DIM), jnp.float32),
            jax.ShapeDtypeStruct((N_PAD, 2), jnp.float32),
        ],
    )(h_pad, W_fc, Wcat_scaled)


def _sc_logits_body(src_h, dst_h, u_h, s1_h, s2_h, wm_h,
                    ne_h, mpart_h,
                    s1_v, s2_v, mmax_v, src_v, dst_v, u_v, ne_v, wm_v,
                    macc_v, mtmp_v, shared_max):
    cid = lax.axis_index("c")
    sid = lax.axis_index("s")
    wid = cid * NS + sid
    base = wid * EW
    pltpu.sync_copy(s1_h, s1_v)
    pltpu.sync_copy(s2_h, s2_v)
    pltpu.sync_copy(wm_h, wm_v)
    pltpu.sync_copy(src_h.at[pl.ds(base, EW)], src_v)
    pltpu.sync_copy(dst_h.at[pl.ds(base, EW)], dst_v)
    pltpu.sync_copy(u_h.at[pl.ds(base, EW)], u_v)
    wm = wm_v[...]

    def init(i, carry):
        mmax_v[pl.ds(i * 16, 16)] = jnp.full((16,), NEG, jnp.float32)
        return carry
    lax.fori_loop(0, N_PAD // 16, init, 0)

    def grp(g, carry):
        sl = pl.ds(g * 16, 16)
        src16 = src_v[sl]
        dst16 = dst_v[sl]
        u16 = u_v[sl]
        a = plsc.load_gather(s1_v, [src16]) + plsc.load_gather(s2_v, [dst16])
        t = a * u16
        ne = wm * jnp.where(t >= 0, t, 0.01 * t)
        ne_v[sl] = ne

        def rtry(r, c2):
            cur = plsc.load_gather(mmax_v, [dst16])
            plsc.store_scatter(mmax_v, [dst16], ne, mask=ne > cur)
            return c2
        lax.fori_loop(0, 6, rtry, 0)
        return carry
    lax.fori_loop(0, EW // 16, grp, 0)
    pltpu.sync_copy(ne_v, ne_h.at[pl.ds(base, EW)])

    pltpu.sync_copy(mmax_v, shared_max.at[sid])
    plsc.subcore_barrier()
    rsl = pl.ds(sid * ROWS_T, ROWS_T)
    pltpu.sync_copy(shared_max.at[0, rsl], macc_v)

    def mrg(t, carry):
        pltpu.sync_copy(shared_max.at[t, rsl], mtmp_v)

        def mx(i, c2):
            isl = pl.ds(i * 16, 16)
            macc_v[isl] = jnp.maximum(macc_v[isl], mtmp_v[isl])
            return c2
        lax.fori_loop(0, ROWS_T // 16, mx, 0)
        return carry
    lax.fori_loop(1, NS, mrg, 0)
    pltpu.sync_copy(macc_v, mpart_h.at[cid, rsl])


def _sc_agg_body(src_h, dst2_h, ne_h, mpart_h, z_h,
                 acc_h, den_h,
                 m_v, src_v, ne_v, dst2_v, u_c0, u_c1, idx0, idx1,
                 rows0, rows1, zden_v, acc_sp, den_sp, sem0, sem1):
    cid = lax.axis_index("c")
    sid = lax.axis_index("s")
    base = jnp.where(cid == 0, sid * EW0, NS * EW0 + sid * EW1)
    npass = jnp.where(cid == 0, EW0 // EWB, EW1 // EWB)

    pltpu.sync_copy(mpart_h.at[0], m_v)

    def mpass(q, carry):
        pltpu.sync_copy(mpart_h.at[1, pl.ds(q * EWB, EWB)], ne_v)

        def mx(i, c2):
            m_v[pl.ds(q * EWB + i * 16, 16)] = jnp.maximum(
                m_v[pl.ds(q * EWB + i * 16, 16)], ne_v[pl.ds(i * 16, 16)])
            return c2
        lax.fori_loop(0, EWB // 16, mx, 0)
        return carry
    lax.fori_loop(0, N_PAD // EWB, mpass, 0)

    def z16(i, carry):
        for j in range(DIM // 16):
            rows0[i, pl.ds(j * 16, 16)] = jnp.zeros((16,), jnp.float32)
        return carry
    lax.fori_loop(0, C2, z16, 0)

    def zden(i, carry):
        zden_v[pl.ds(i * 16, 16)] = jnp.zeros((16,), jnp.float32)
        return carry
    lax.fori_loop(0, ROWS_T // 16, zden, 0)

    def zacc(i, carry):
        pltpu.sync_copy(rows0, acc_sp.at[pl.ds(sid * ROWS_T + i * C2, C2)])
        return carry
    lax.fori_loop(0, ROWS_T // C2, zacc, 0)
    pltpu.sync_copy(zden_v, den_sp.at[pl.ds(sid * ROWS_T, ROWS_T)])
    plsc.subcore_barrier()

    bufs = ((u_c0, rows0, sem0, idx0), (u_c1, rows1, sem1, idx1))

    def issue_gather(c, b):
        _, rows_b, sem_b, _ = bufs[b]
        pltpu.async_copy(z_h.at[src_v.at[pl.ds(c * C2, C2)]], rows_b, sem_b)

    def pas(p, carry):
        pbase = pl.multiple_of(base + p * EWB, EWB)
        prow = pl.multiple_of(pbase // C2, CPP)
        pltpu.sync_copy(src_h.at[pl.ds(pbase, EWB)], src_v)
        pltpu.sync_copy(ne_h.at[pl.ds(pbase, EWB)], ne_v)
        pltpu.sync_copy(dst2_h.at[pl.ds(prow, CPP)], dst2_v)
        for b in (0, 1):
            issue_gather(b, b)

        def outer(c2i, carry2):
            for b in (0, 1):
                u_b, rows_b, sem_b, idx_b = bufs[b]
                c = 2 * c2i + b

                def grp(g, c3):
                    sl = pl.ds(g * 16, 16)
                    dst16 = dst2_v[c, sl]
                    idx_b[sl] = dst16
                    m16 = plsc.load_gather(m_v, [dst16])
                    u_b[sl] = jnp.exp(ne_v[pl.ds(c * C2 + g * 16, 16)] - m16)
                    return c3
                lax.fori_loop(0, C2 // 16, grp, 0)

                pltpu.make_async_copy(z_h.at[pl.ds(0, C2)], rows_b,
                                      sem_b).wait()

                def scale(e, c3):
                    ue = plsc.load_gather(u_b,
                                          [jnp.zeros((16,), jnp.int32) + e])
                    for j in range(DIM // 16):
                        jsl = pl.ds(j * 16, 16)
                        rows_b[e, jsl] = rows_b[e, jsl] * ue
                    return c3
                lax.fori_loop(0, C2, scale, 0)

                pltpu.sync_copy(u_b, den_sp.at[idx_b], add=True)
                pltpu.sync_copy(rows_b, acc_sp.at[idx_b], add=True)

                @pl.when(c + 2 < CPP)
                def _():
                    issue_gather(c + 2, b)
            return carry2
        lax.fori_loop(0, CPP // 2, outer, 0)
        return carry
    lax.fori_loop(0, npass, pas, 0)
    plsc.subcore_barrier()

    rsl = pl.ds(sid * ROWS_T, ROWS_T)
    pltpu.sync_copy(acc_sp.at[rsl], acc_h.at[cid, rsl])
    pltpu.sync_copy(den_sp.at[rsl], den_h.at[cid, rsl])


def _tc_merge_body(acc_ref, den_ref, o_ref):
    num = acc_ref[0] + acc_ref[1]
    den = den_ref[0] + den_ref[1]
    d = jnp.where(den > 0, den, 1.0)
    o_ref[...] = _leaky(num / d)


def _tc_merge(acc, den3):
    return pl.pallas_call(
        _tc_merge_body,
        grid=(N_PAD // BLK,),
        in_specs=[
            pl.BlockSpec((NC, BLK, DIM), lambda i: (0, i, 0)),
            pl.BlockSpec((NC, BLK, 1), lambda i: (0, i, 0)),
        ],
        out_specs=pl.BlockSpec((BLK, DIM), lambda i: (i, 0)),
        out_shape=jax.ShapeDtypeStruct((N_PAD, DIM), jnp.float32),
    )(acc, den3)


def kernel(h, edge_index, edge_attr, W_fc, W_attn, W_edge, W_m):
    E = edge_index.shape[1]
    src = edge_index[0].astype(jnp.int32)
    dst = edge_index[1].astype(jnp.int32)
    u = edge_attr[:, 0]
    pad = E_ALLOC - E
    src = jnp.concatenate([src, jnp.zeros((pad,), jnp.int32)])
    pad_dst = N_NODES + (jnp.arange(pad, dtype=jnp.int32) % (N_PAD - N_NODES))
    dst = jnp.concatenate([dst, pad_dst])
    u = jnp.concatenate([u, jnp.zeros((pad,), jnp.float32)])

    h_pad = jnp.pad(h, ((0, N_PAD - N_NODES), (0, 0)))
    Wcat_scaled = W_attn.reshape(2, DIM) * W_edge[0, 0]
    z, s = _tc_node_transform(h_pad, W_fc, Wcat_scaled)
    s1 = s[:, 0] + 0.0
    s2 = s[:, 1] + 0.0
    wm16 = jnp.full((16,), W_m[0, 0], jnp.float32)

    mesh = plsc.VectorSubcoreMesh(core_axis_name="c", subcore_axis_name="s")

    sc_params = pltpu.CompilerParams(needs_layout_passes=False)
    sc1 = pl.kernel(
        _sc_logits_body,
        mesh=mesh,
        compiler_params=sc_params,
        out_type=[
            jax.ShapeDtypeStruct((E_ALLOC,), jnp.float32),
            jax.ShapeDtypeStruct((NC, N_PAD), jnp.float32),
        ],
        scratch_types=[
            pltpu.VMEM((N_PAD,), jnp.float32),
            pltpu.VMEM((N_PAD,), jnp.float32),
            pltpu.VMEM((N_PAD,), jnp.float32),
            pltpu.VMEM((EW,), jnp.int32),
            pltpu.VMEM((EW,), jnp.int32),
            pltpu.VMEM((EW,), jnp.float32),
            pltpu.VMEM((EW,), jnp.float32),
            pltpu.VMEM((16,), jnp.float32),
            pltpu.VMEM((ROWS_T,), jnp.float32),
            pltpu.VMEM((ROWS_T,), jnp.float32),
            pltpu.VMEM_SHARED((NS, N_PAD), jnp.float32),
        ],
    )
    ne, mpart = sc1(src, dst, u, s1, s2, wm16)

    sc2 = pl.kernel(
        _sc_agg_body,
        mesh=mesh,
        compiler_params=sc_params,
        out_type=[
            jax.ShapeDtypeStruct((NC, N_PAD, DIM), jnp.float32),
            jax.ShapeDtypeStruct((NC, N_PAD), jnp.float32),
        ],
        scratch_types=[
            pltpu.VMEM((N_PAD,), jnp.float32),
            pltpu.VMEM((EWB,), jnp.int32),
            pltpu.VMEM((EWB,), jnp.float32),
            pltpu.VMEM((CPP, C2), jnp.int32),
            pltpu.VMEM((C2,), jnp.float32),
            pltpu.VMEM((C2,), jnp.float32),
            pltpu.VMEM((C2,), jnp.int32),
            pltpu.VMEM((C2,), jnp.int32),
            pltpu.VMEM((C2, DIM), jnp.float32),
            pltpu.VMEM((C2, DIM), jnp.float32),
            pltpu.VMEM((ROWS_T,), jnp.float32),
            pltpu.VMEM_SHARED((N_PAD, DIM), jnp.float32),
            pltpu.VMEM_SHARED((N_PAD,), jnp.float32),
            pltpu.SemaphoreType.DMA,
            pltpu.SemaphoreType.DMA,
        ],
    )
    dst2 = dst.reshape(E_ALLOC // C2, C2)
    acc, den = sc2(src, dst2, ne, mpart, z)

    out = _tc_merge(acc, den.reshape(NC, N_PAD, 1))
    return out[:N_NODES]
